# baseline (speedup 1.0000x reference)
import functools

import jax
import jax.numpy as jnp
from jax import lax
from jax.experimental import pallas as pl
from jax.experimental.pallas import tpu as pltpu
from jax.experimental.pallas import tpu_sc as plsc

B = 16384
EMB = 64
L = 16
N_REL = 100
N_NODES = 1000000

_info = plsc.get_sparse_core_info()
_NC, _NS = _info.num_cores, _info.num_subcores
NW = _NC * _NS
BPW = B // NW
NCHUNK = BPW // L
NGRP = BPW // 128
CPG = 128 // L

_mesh = plsc.VectorSubcoreMesh(core_axis_name="c", subcore_axis_name="s")


@functools.partial(
    pl.kernel,
    mesh=_mesh,
    compiler_params=pltpu.CompilerParams(needs_layout_passes=False),
    out_type=jax.ShapeDtypeStruct((B,), jnp.float32),
    scratch_types=[
        pltpu.VMEM((NGRP, 128), jnp.int32),
        pltpu.VMEM((NGRP, 128), jnp.int32),
        pltpu.VMEM((NCHUNK, L), jnp.int32),
        pltpu.VMEM((NGRP, 128), jnp.int32),
        pltpu.VMEM((NGRP, 128), jnp.int32),
        pltpu.VMEM((NCHUNK, L), jnp.int32),
        pltpu.VMEM((NCHUNK, L), jnp.int32),
        pltpu.VMEM((128, 128), jnp.float32),
        pltpu.VMEM((128, 128), jnp.float32),
        pltpu.VMEM((128, 128), jnp.float32),
        pltpu.VMEM((128, 128), jnp.float32),
        pltpu.VMEM((N_REL, EMB), jnp.float32),
        pltpu.VMEM((BPW,), jnp.float32),
        pltpu.SemaphoreType.DMA,
        pltpu.SemaphoreType.DMA,
        pltpu.SemaphoreType.DMA,
        pltpu.SemaphoreType.DMA,
    ],
)
def _lp_kernel(s_hbm, p_hbm, o_hbm, node_hbm, rel_hbm, out_hbm,
               sidx_v, oidx_v, pidx_v, srid_v, orid_v, scol_v, ocol_v,
               sbuf0, sbuf1, obuf0, obuf1, rel_l, out_v,
               sem_s0, sem_s1, sem_o0, sem_o1):
    wid = lax.axis_index("s") * _NC + lax.axis_index("c")

    pltpu.sync_copy(s_hbm.at[wid], sidx_v)
    pltpu.sync_copy(o_hbm.at[wid], oidx_v)
    pltpu.sync_copy(p_hbm.at[wid], pidx_v)

    for c in range(NCHUNK):
        g, k = divmod(c, CPG)
        sl = pl.ds(k * L, L)
        sv = sidx_v[g, sl]
        srid_v[g, sl] = sv >> 1
        scol_v[c, :] = (sv & 1) << 6
        ov = oidx_v[g, sl]
        orid_v[g, sl] = ov >> 1
        ocol_v[c, :] = (ov & 1) << 6

    sbufs = (sbuf0, sbuf1)
    obufs = (obuf0, obuf1)
    ssems = (sem_s0, sem_s1)
    osems = (sem_o0, sem_o1)

    def issue(g):
        hs = pltpu.async_copy(node_hbm.at[srid_v.at[g]],
                              sbufs[g % 2], ssems[g % 2])
        ho = pltpu.async_copy(node_hbm.at[orid_v.at[g]],
                              obufs[g % 2], osems[g % 2])
        return hs, ho

    pending = issue(0)
    pltpu.sync_copy(rel_hbm, rel_l)

    lane = lax.iota(jnp.int32, L)

    for g in range(NGRP):
        nxt = issue(g + 1) if g + 1 < NGRP else None
        pending[0].wait()
        pending[1].wait()
        pending = nxt
        sb, ob = sbufs[g % 2], obufs[g % 2]

        def chunk_body(lc, carry, g=g, sb=sb, ob=ob):
            c = g * CPG + lc
            rows = lc * L + lane
            scol = scol_v[c, :]
            ocol = ocol_v[c, :]
            pvec = pidx_v[c, :]
            acc = jnp.zeros((L,), jnp.float32)
            for d in range(EMB):
                sv = plsc.load_gather(sb, [rows, scol + d])
                ov = plsc.load_gather(ob, [rows, ocol + d])
                pv = plsc.load_gather(rel_l, [pvec, lane * 0 + d])
                acc = acc + sv * pv * ov
            out_v[pl.ds(c * L, L)] = 1.0 / (1.0 + jnp.exp(-acc))
            return carry

        lax.fori_loop(0, CPG, chunk_body, 0)

    pltpu.sync_copy(out_v, out_hbm.at[pl.ds(wid * BPW, BPW)])


def kernel(s_idx, p_idx, o_idx, node_emb, rel_emb):
    s3 = s_idx.reshape(NW, NGRP, 128)
    o3 = o_idx.reshape(NW, NGRP, 128)
    p3 = p_idx.reshape(NW, NCHUNK, L)
    node2 = node_emb.reshape(N_NODES // 2, 128)
    return _lp_kernel(s3, p3, o3, node2, rel_emb)


# speedup vs baseline: 2.2929x; 2.2929x over previous
import functools

import jax
import jax.numpy as jnp
from jax import lax
from jax.experimental import pallas as pl
from jax.experimental.pallas import tpu as pltpu
from jax.experimental.pallas import tpu_sc as plsc

B = 16384
EMB = 64
L = 16
N_REL = 100
N_NODES = 1000000

_info = plsc.get_sparse_core_info()
_NC, _NS = _info.num_cores, _info.num_subcores
NW = _NC * _NS
BPW = B // NW
NCHUNK = BPW // L
NGRP = BPW // 128
CPG = 128 // L

_mesh = plsc.VectorSubcoreMesh(core_axis_name="c", subcore_axis_name="s")


@functools.partial(
    pl.kernel,
    mesh=_mesh,
    compiler_params=pltpu.CompilerParams(needs_layout_passes=False),
    out_type=jax.ShapeDtypeStruct((B,), jnp.float32),
    scratch_types=[
        pltpu.VMEM((NGRP, 128), jnp.int32),
        pltpu.VMEM((NGRP, 128), jnp.int32),
        pltpu.VMEM((NCHUNK, L), jnp.int32),
        pltpu.VMEM((16, 8, EMB), jnp.float32),
        pltpu.VMEM((16, 8, EMB), jnp.float32),
        pltpu.VMEM((16, 8, EMB), jnp.float32),
        pltpu.VMEM((16, 8, EMB), jnp.float32),
        pltpu.VMEM((N_REL, EMB), jnp.float32),
        pltpu.VMEM((BPW,), jnp.float32),
        pltpu.SemaphoreType.DMA,
        pltpu.SemaphoreType.DMA,
        pltpu.SemaphoreType.DMA,
        pltpu.SemaphoreType.DMA,
    ],
)
def _lp_kernel(s_hbm, p_hbm, o_hbm, node_hbm, rel_hbm, out_hbm,
               sidx_v, oidx_v, pidx_v,
               sbuf0, sbuf1, obuf0, obuf1, rel_l, out_v,
               sem_s0, sem_s1, sem_o0, sem_o1):
    wid = lax.axis_index("s") * _NC + lax.axis_index("c")

    pltpu.sync_copy(s_hbm.at[wid], sidx_v)
    pltpu.sync_copy(o_hbm.at[wid], oidx_v)
    pltpu.sync_copy(p_hbm.at[wid], pidx_v)

    sbufs = (sbuf0, sbuf1)
    obufs = (obuf0, obuf1)
    ssems = (sem_s0, sem_s1)
    osems = (sem_o0, sem_o1)

    def issue(g):
        sb, ob = sbufs[g % 2], obufs[g % 2]
        sem_s, sem_o = ssems[g % 2], osems[g % 2]

        def dma_body(k, carry):
            sv = sidx_v[g, pl.ds(k * L, L)]
            ov = oidx_v[g, pl.ds(k * L, L)]
            for j in range(L):
                i = k * L + j
                rs = sv[j]
                pltpu.async_copy(
                    node_hbm.at[pl.ds(rs >> 3, 1), pl.ds(rs & 7, 1)],
                    sb.at[pl.ds(i >> 3, 1), pl.ds(i & 7, 1)], sem_s)
                ro = ov[j]
                pltpu.async_copy(
                    node_hbm.at[pl.ds(ro >> 3, 1), pl.ds(ro & 7, 1)],
                    ob.at[pl.ds(i >> 3, 1), pl.ds(i & 7, 1)], sem_o)
            return carry

        lax.fori_loop(0, CPG, dma_body, 0)

    def wait(par):
        pltpu.make_async_copy(node_hbm.at[pl.ds(0, 16)],
                              sbufs[par], ssems[par]).wait()
        pltpu.make_async_copy(node_hbm.at[pl.ds(0, 16)],
                              obufs[par], osems[par]).wait()

    issue(0)
    pltpu.sync_copy(rel_hbm, rel_l)

    lane = lax.iota(jnp.int32, L)

    for g in range(NGRP):
        if g + 1 < NGRP:
            issue(g + 1)
        wait(g % 2)
        sb, ob = sbufs[g % 2], obufs[g % 2]

        def chunk_body(lc, carry, g=g, sb=sb, ob=ob):
            c = g * CPG + lc
            rows = lc * L + lane
            tq = rows >> 3
            tr = rows & 7
            pvec = pidx_v[c, :]
            acc = jnp.zeros((L,), jnp.float32)
            for d in range(EMB):
                sv = plsc.load_gather(sb, [tq, tr, lane * 0 + d])
                ov = plsc.load_gather(ob, [tq, tr, lane * 0 + d])
                pv = plsc.load_gather(rel_l, [pvec, lane * 0 + d])
                acc = acc + sv * pv * ov
            out_v[pl.ds(c * L, L)] = 1.0 / (1.0 + jnp.exp(-acc))
            return carry

        lax.fori_loop(0, CPG, chunk_body, 0)

    pltpu.sync_copy(out_v, out_hbm.at[pl.ds(wid * BPW, BPW)])


def kernel(s_idx, p_idx, o_idx, node_emb, rel_emb):
    s3 = s_idx.reshape(NW, NGRP, 128)
    o3 = o_idx.reshape(NW, NGRP, 128)
    p3 = p_idx.reshape(NW, NCHUNK, L)
    node3 = node_emb.reshape(N_NODES // 8, 8, EMB)
    return _lp_kernel(s3, p3, o3, node3, rel_emb)
